# baseline (speedup 1.0000x reference)
import functools

import jax
import jax.numpy as jnp
from jax import lax
from jax.experimental import pallas as pl
from jax.experimental.pallas import tpu as pltpu
from jax.experimental.pallas import tpu_sc as plsc

_NC, _NS, _L = 2, 16, 16
_NW = _NC * _NS
_D = 128
_CTX = 20
_NNEG = 100
_K = 128


def _sc_scores(iword, idx_all, in_W, out_W):
  B = iword.shape[0]
  bpw = B // _NW
  nbuf = 4
  nd = _D // _L
  ng = _K // _L

  mesh = plsc.VectorSubcoreMesh(core_axis_name="c", subcore_axis_name="s")

  @functools.partial(
      pl.kernel,
      out_type=jax.ShapeDtypeStruct((B, _K), jnp.float32),
      mesh=mesh,
      scratch_types=[
          pltpu.VMEM((bpw,), jnp.int32),
          pltpu.VMEM((bpw, _K), jnp.int32),
          pltpu.VMEM((bpw, _D), jnp.float32),
          pltpu.VMEM((nbuf, _K, _D), jnp.float32),
          pltpu.VMEM((bpw, _K), jnp.float32),
          pltpu.SemaphoreType.DMA,
          pltpu.SemaphoreType.DMA,
          pltpu.SemaphoreType.DMA,
          pltpu.SemaphoreType.DMA,
          pltpu.SemaphoreType.DMA,
      ],
  )
  def sc_kernel(iword_hbm, idx_hbm, inw_hbm, outw_hbm, out_hbm,
                iw_v, idx_v, ivec_v, rows_v, sc_v,
                sem0, sem1, sem2, sem3, semi):
    wid = lax.axis_index("s") * _NC + lax.axis_index("c")
    base = wid * bpw
    pltpu.sync_copy(iword_hbm.at[pl.ds(base, bpw)], iw_v)
    pltpu.sync_copy(idx_hbm.at[pl.ds(base, bpw), :], idx_v)
    pltpu.async_copy(inw_hbm.at[iw_v], ivec_v, semi).wait()

    lane = lax.iota(jnp.int32, _L)
    dnums = lax.GatherDimensionNumbers(
        offset_dims=(), collapsed_slice_dims=(0,), start_index_map=(0,))
    rot_idx = [(lane + k) % _L for k in (8, 4, 2, 1)]

    def hsum(v):
      for idx in rot_idx:
        v = v + lax.gather(v, idx[:, None], dnums, slice_sizes=(1,),
                           mode=lax.GatherScatterMode.PROMISE_IN_BOUNDS)
      return v

    def compute(b, buf):
      iv = [ivec_v[b, pl.ds(k * _L, _L)] for k in range(nd)]
      for r in range(ng):
        res = jnp.zeros((_L,), jnp.float32)
        for j in range(_L):
          row = r * _L + j
          acc = rows_v[buf, row, pl.ds(0, _L)] * iv[0]
          for k in range(1, nd):
            acc = acc + rows_v[buf, row, pl.ds(k * _L, _L)] * iv[k]
          res = jnp.where(lane == j, hsum(acc), res)
        sc_v[b, pl.ds(r * _L, _L)] = res

    def start(b, buf, sem):
      pltpu.async_copy(outw_hbm.at[idx_v.at[b]], rows_v.at[buf], sem)

    def wait(buf, sem):
      pltpu.make_async_copy(outw_hbm.at[idx_v.at[0]], rows_v.at[buf],
                            sem).wait()

    sems = (sem0, sem1, sem2, sem3)
    last = bpw - 1
    for u in range(nbuf):
      start(u, u, sems[u])

    def body(i, _):
      g = nbuf * i
      for u in range(nbuf):
        wait(u, sems[u])
        compute(g + u, u)
        start(jnp.minimum(g + u + nbuf, last), u, sems[u])
      return 0

    lax.fori_loop(0, bpw // nbuf, body, 0)
    for u in range(nbuf):
      wait(u, sems[u])
    pltpu.sync_copy(sc_v, out_hbm.at[pl.ds(base, bpw), :])

  return sc_kernel(iword, idx_all, in_W, out_W)


def _log_sigmoid(x):
  return jnp.minimum(x, 0.0) - jnp.log1p(jnp.exp(-jnp.abs(x)))


def _tc_finish(scores, n_ctx, n_neg):
  B, K = scores.shape

  def body(s_ref, so_ref, sn_ref, loss_ref):
    s = s_ref[...]
    col = lax.broadcasted_iota(jnp.int32, s.shape, 1)
    ls_p = _log_sigmoid(s)
    ls_n = _log_sigmoid(-s)
    o = jnp.sum(jnp.where(col < n_ctx, ls_p, 0.0), axis=1) / n_ctx
    n = jnp.sum(jnp.where((col >= n_ctx) & (col < n_ctx + n_neg), ls_n, 0.0),
                axis=1) / n_ctx
    so_ref[...] = o
    sn_ref[...] = n
    loss_ref[...] = jnp.full((1, 1), -1.0) * jnp.mean(o + n)

  return pl.pallas_call(
      body,
      out_shape=(
          jax.ShapeDtypeStruct((B,), jnp.float32),
          jax.ShapeDtypeStruct((B,), jnp.float32),
          jax.ShapeDtypeStruct((1, 1), jnp.float32),
      ),
  )(scores)


def kernel(iword, owords, nwords, in_W, out_W):
  B = iword.shape[0]
  pad = jnp.zeros((B, _K - _CTX - _NNEG), jnp.int32)
  idx_all = jnp.concatenate(
      [owords.astype(jnp.int32), nwords.astype(jnp.int32), pad], axis=1)
  scores = _sc_scores(iword.astype(jnp.int32), idx_all, in_W, out_W)
  score_o, score_n, loss = _tc_finish(scores, _CTX, _NNEG)
  return (loss[0, 0], score_o, score_n)


# speedup vs baseline: 1.6977x; 1.6977x over previous
import functools

import jax
import jax.numpy as jnp
from jax import lax
from jax.experimental import pallas as pl
from jax.experimental.pallas import tpu as pltpu
from jax.experimental.pallas import tpu_sc as plsc

_NC, _NS, _L = 2, 16, 16
_NW = _NC * _NS
_D = 128
_DW = _D // 2
_CTX = 20
_NNEG = 100
_KR = _CTX + _NNEG
_KO = 128


def _sc_scores(iword, idx_all, in_W, out_pk):
  B = iword.shape[0]
  bpw = B // _NW
  nbuf = 4
  nck = _D // (2 * _L)

  mesh = plsc.VectorSubcoreMesh(core_axis_name="c", subcore_axis_name="s")

  @functools.partial(
      pl.kernel,
      out_type=jax.ShapeDtypeStruct((B, _KO), jnp.float32),
      mesh=mesh,
      compiler_params=pltpu.CompilerParams(use_tc_tiling_on_sc=False),
      scratch_types=[
          pltpu.VMEM((bpw,), jnp.int32),
          pltpu.VMEM((bpw, _KR), jnp.int32),
          pltpu.VMEM((bpw, _D), jnp.float32),
          pltpu.VMEM((nbuf, _KR, _DW), jnp.int32),
          pltpu.VMEM((bpw, _KO), jnp.float32),
          pltpu.SemaphoreType.DMA,
          pltpu.SemaphoreType.DMA,
          pltpu.SemaphoreType.DMA,
          pltpu.SemaphoreType.DMA,
          pltpu.SemaphoreType.DMA,
      ],
  )
  def sc_kernel(iword_hbm, idx_hbm, inw_hbm, outw_hbm, out_hbm,
                iw_v, idx_v, ivec_v, rows_v, sc_v,
                sem0, sem1, sem2, sem3, semi):
    wid = lax.axis_index("s") * _NC + lax.axis_index("c")
    base = wid * bpw
    pltpu.sync_copy(iword_hbm.at[pl.ds(base, bpw)], iw_v)
    pltpu.sync_copy(idx_hbm.at[pl.ds(base, bpw), :], idx_v)
    pltpu.async_copy(inw_hbm.at[iw_v], ivec_v, semi).wait()

    lane = lax.iota(jnp.int32, _L)
    dnums = lax.GatherDimensionNumbers(
        offset_dims=(), collapsed_slice_dims=(0,), start_index_map=(0,))

    def perm(v, idx):
      return lax.gather(v, idx[:, None], dnums, slice_sizes=(1,),
                        mode=lax.GatherScatterMode.PROMISE_IN_BOUNDS)

    rot_idx = [(lane + k) % _L for k in (8, 4, 2, 1)]
    pidx_e = (2 * lane) % _L
    pidx_o = (2 * lane + 1) % _L
    lane_lo = lane < 8
    msk_hi = jnp.full((_L,), -65536, jnp.int32)

    def hsum(v):
      for idx in rot_idx:
        v = v + perm(v, idx)
      return v

    def compute(b, buf):
      ive, ivo = [], []
      for c in range(nck):
        a = ivec_v[b, pl.ds(c * 32, _L)]
        bb = ivec_v[b, pl.ds(c * 32 + _L, _L)]
        ive.append(jnp.where(lane_lo, perm(a, pidx_e), perm(bb, pidx_e)))
        ivo.append(jnp.where(lane_lo, perm(a, pidx_o), perm(bb, pidx_o)))

      def dot_row(row):
        acc = jnp.zeros((_L,), jnp.float32)
        for c in range(nck):
          v = rows_v[buf, row, pl.ds(c * _L, _L)]
          lo = lax.bitcast_convert_type(v << 16, jnp.float32)
          hi = lax.bitcast_convert_type(v & msk_hi, jnp.float32)
          acc = acc + lo * ive[c] + hi * ivo[c]
        return acc

      for r in range(_KR // _L):
        res = jnp.zeros((_L,), jnp.float32)
        for j in range(_L):
          res = jnp.where(lane == j, hsum(dot_row(r * _L + j)), res)
        sc_v[b, pl.ds(r * _L, _L)] = res
      res = jnp.zeros((_L,), jnp.float32)
      for j in range(_KR % _L):
        res = jnp.where(lane == j, hsum(dot_row(112 + j)), res)
      sc_v[b, pl.ds(112, _L)] = res

    def start(b, buf, sem):
      pltpu.async_copy(outw_hbm.at[idx_v.at[b]], rows_v.at[buf], sem)

    def wait(buf, sem):
      pltpu.make_async_copy(outw_hbm.at[idx_v.at[0]], rows_v.at[buf],
                            sem).wait()

    sems = (sem0, sem1, sem2, sem3)
    last = bpw - 1
    for u in range(nbuf):
      start(u, u, sems[u])

    def body(i, _):
      g = nbuf * i
      for u in range(nbuf):
        wait(u, sems[u])
        compute(g + u, u)
        start(jnp.minimum(g + u + nbuf, last), u, sems[u])
      return 0

    lax.fori_loop(0, bpw // nbuf, body, 0)
    for u in range(nbuf):
      wait(u, sems[u])
    pltpu.sync_copy(sc_v, out_hbm.at[pl.ds(base, bpw), :])

  return sc_kernel(iword, idx_all, in_W, out_pk)


def _log_sigmoid(x):
  return jnp.minimum(x, 0.0) - jnp.log1p(jnp.exp(-jnp.abs(x)))


def _tc_finish(scores, n_ctx, n_neg):
  B, K = scores.shape

  def body(s_ref, so_ref, sn_ref, loss_ref):
    s = s_ref[...]
    col = lax.broadcasted_iota(jnp.int32, s.shape, 1)
    ls_p = _log_sigmoid(s)
    ls_n = _log_sigmoid(-s)
    o = jnp.sum(jnp.where(col < n_ctx, ls_p, 0.0), axis=1) / n_ctx
    n = jnp.sum(jnp.where((col >= n_ctx) & (col < n_ctx + n_neg), ls_n, 0.0),
                axis=1) / n_ctx
    so_ref[...] = o
    sn_ref[...] = n
    loss_ref[...] = jnp.full((1, 1), -1.0) * jnp.mean(o + n)

  return pl.pallas_call(
      body,
      out_shape=(
          jax.ShapeDtypeStruct((B,), jnp.float32),
          jax.ShapeDtypeStruct((B,), jnp.float32),
          jax.ShapeDtypeStruct((1, 1), jnp.float32),
      ),
  )(scores)


def kernel(iword, owords, nwords, in_W, out_W):
  V = out_W.shape[0]
  idx_all = jnp.concatenate(
      [owords.astype(jnp.int32), nwords.astype(jnp.int32)], axis=1)
  out_pk = lax.bitcast_convert_type(
      out_W.astype(jnp.bfloat16).reshape(V, _DW, 2), jnp.int32)
  scores = _sc_scores(iword.astype(jnp.int32), idx_all, in_W, out_pk)
  score_o, score_n, loss = _tc_finish(scores, _CTX, _NNEG)
  return (loss[0, 0], score_o, score_n)


# speedup vs baseline: 3.1321x; 1.8449x over previous
import functools

import jax
import jax.numpy as jnp
from jax import lax
from jax.experimental import pallas as pl
from jax.experimental.pallas import tpu as pltpu
from jax.experimental.pallas import tpu_sc as plsc

_NC, _NS, _L = 2, 16, 16
_NW = _NC * _NS
_D = 128
_DW = _D // 2
_CTX = 20
_NNEG = 100
_KR = _CTX + _NNEG
_KO = 128


def _sc_scores(iword, idx_all, in_W, out_pk):
  B = iword.shape[0]
  bpw = B // _NW
  nbuf = 4
  nck = _D // (2 * _L)

  mesh = plsc.VectorSubcoreMesh(core_axis_name="c", subcore_axis_name="s")

  @functools.partial(
      pl.kernel,
      out_type=jax.ShapeDtypeStruct((B, _KO), jnp.float32),
      mesh=mesh,
      compiler_params=pltpu.CompilerParams(use_tc_tiling_on_sc=False),
      scratch_types=[
          pltpu.VMEM((bpw,), jnp.int32),
          pltpu.VMEM((bpw, _KR), jnp.int32),
          pltpu.VMEM((bpw, _D), jnp.float32),
          pltpu.VMEM((nbuf, _KR, _DW), jnp.int32),
          pltpu.VMEM((bpw, _KO), jnp.float32),
          pltpu.SemaphoreType.DMA,
          pltpu.SemaphoreType.DMA,
          pltpu.SemaphoreType.DMA,
          pltpu.SemaphoreType.DMA,
          pltpu.SemaphoreType.DMA,
      ],
  )
  def sc_kernel(iword_hbm, idx_hbm, inw_hbm, outw_hbm, out_hbm,
                iw_v, idx_v, ivec_v, rows_v, sc_v,
                sem0, sem1, sem2, sem3, semi):
    wid = lax.axis_index("s") * _NC + lax.axis_index("c")
    base = wid * bpw
    pltpu.sync_copy(iword_hbm.at[pl.ds(base, bpw)], iw_v)
    pltpu.sync_copy(idx_hbm.at[pl.ds(base, bpw), :], idx_v)
    pltpu.async_copy(inw_hbm.at[iw_v], ivec_v, semi).wait()

    lane = lax.iota(jnp.int32, _L)
    dnums = lax.GatherDimensionNumbers(
        offset_dims=(), collapsed_slice_dims=(0,), start_index_map=(0,))

    def perm(v, idx):
      return lax.gather(v, idx[:, None], dnums, slice_sizes=(1,),
                        mode=lax.GatherScatterMode.PROMISE_IN_BOUNDS)

    rot_idx = [(lane + k) % _L for k in (8, 4, 2, 1)]
    msk_hi = jnp.full((_L,), -65536, jnp.int32)

    def hsum(v):
      for idx in rot_idx:
        v = v + perm(v, idx)
      return v

    def compute(b, buf):
      iv = [ivec_v[b, pl.ds(k * _L, _L)] for k in range(2 * nck)]

      def dot_row(row):
        acc = jnp.zeros((_L,), jnp.float32)
        for c in range(nck):
          v = rows_v[buf, row, pl.ds(c * _L, _L)]
          lo = lax.bitcast_convert_type(v << 16, jnp.float32)
          hi = lax.bitcast_convert_type(v & msk_hi, jnp.float32)
          acc = acc + lo * iv[c] + hi * iv[c + nck]
        return acc

      for r in range(_KR // _L):
        res = jnp.zeros((_L,), jnp.float32)
        for j in range(_L):
          res = jnp.where(lane == j, hsum(dot_row(r * _L + j)), res)
        sc_v[b, pl.ds(r * _L, _L)] = res
      res = jnp.zeros((_L,), jnp.float32)
      for j in range(_KR % _L):
        res = jnp.where(lane == j, hsum(dot_row(112 + j)), res)
      sc_v[b, pl.ds(112, _L)] = res

    def start(b, buf, sem):
      pltpu.async_copy(outw_hbm.at[idx_v.at[b]], rows_v.at[buf], sem)

    def wait(buf, sem):
      pltpu.make_async_copy(outw_hbm.at[idx_v.at[0]], rows_v.at[buf],
                            sem).wait()

    sems = (sem0, sem1, sem2, sem3)
    last = bpw - 1
    for u in range(nbuf):
      start(u, u, sems[u])

    def body(i, _):
      g = nbuf * i
      for u in range(nbuf):
        wait(u, sems[u])
        compute(g + u, u)
        start(jnp.minimum(g + u + nbuf, last), u, sems[u])
      return 0

    lax.fori_loop(0, bpw // nbuf, body, 0)
    for u in range(nbuf):
      wait(u, sems[u])
    pltpu.sync_copy(sc_v, out_hbm.at[pl.ds(base, bpw), :])

  return sc_kernel(iword, idx_all, in_W, out_pk)


def _tc_pack(out_W):
  V, D = out_W.shape
  RB = 2000

  def body(x_ref, o_ref):
    xb = x_ref[...].astype(jnp.bfloat16)
    lo = lax.bitcast_convert_type(xb[:, :_DW], jnp.uint16).astype(jnp.uint32)
    hi = lax.bitcast_convert_type(xb[:, _DW:], jnp.uint16).astype(jnp.uint32)
    o_ref[...] = lax.bitcast_convert_type(lo | (hi << 16), jnp.int32)

  return pl.pallas_call(
      body,
      grid=(V // RB,),
      in_specs=[pl.BlockSpec((RB, D), lambda i: (i, 0))],
      out_specs=pl.BlockSpec((RB, _DW), lambda i: (i, 0)),
      out_shape=jax.ShapeDtypeStruct((V, _DW), jnp.int32),
  )(out_W)


def _log_sigmoid(x):
  return jnp.minimum(x, 0.0) - jnp.log1p(jnp.exp(-jnp.abs(x)))


def _tc_finish(scores, n_ctx, n_neg):
  B, K = scores.shape

  def body(s_ref, so_ref, sn_ref, loss_ref):
    s = s_ref[...]
    col = lax.broadcasted_iota(jnp.int32, s.shape, 1)
    ls_p = _log_sigmoid(s)
    ls_n = _log_sigmoid(-s)
    o = jnp.sum(jnp.where(col < n_ctx, ls_p, 0.0), axis=1) / n_ctx
    n = jnp.sum(jnp.where((col >= n_ctx) & (col < n_ctx + n_neg), ls_n, 0.0),
                axis=1) / n_ctx
    so_ref[...] = o
    sn_ref[...] = n
    loss_ref[...] = jnp.full((1, 1), -1.0) * jnp.mean(o + n)

  return pl.pallas_call(
      body,
      out_shape=(
          jax.ShapeDtypeStruct((B,), jnp.float32),
          jax.ShapeDtypeStruct((B,), jnp.float32),
          jax.ShapeDtypeStruct((1, 1), jnp.float32),
      ),
  )(scores)


def kernel(iword, owords, nwords, in_W, out_W):
  idx_all = jnp.concatenate(
      [owords.astype(jnp.int32), nwords.astype(jnp.int32)], axis=1)
  out_pk = _tc_pack(out_W)
  scores = _sc_scores(iword.astype(jnp.int32), idx_all, in_W, out_pk)
  score_o, score_n, loss = _tc_finish(scores, _CTX, _NNEG)
  return (loss[0, 0], score_o, score_n)
